# baseline (speedup 1.0000x reference)
import functools

import jax
import jax.numpy as jnp
from jax import lax
from jax.experimental import pallas as pl
from jax.experimental.pallas import tpu as pltpu
from jax.experimental.pallas import tpu_sc as plsc

B = 4096
D_IN = 1024
D_HID = 1000
D_OUT = 1000
D_PAD = 1024
BLK = 256
NBLK = B // BLK + 2
NSLOT = NBLK * BLK
NSMALL = 128
NW = 32


def _routing_metadata(question_type_output):
    q = question_type_output
    pred = jnp.argmax(q, axis=1).astype(jnp.int32)
    onehot = (pred[:, None] == jnp.arange(3, dtype=jnp.int32)[None, :])
    ranks = jnp.cumsum(onehot.astype(jnp.int32), axis=0)
    counts = ranks[-1]
    rank = jnp.sum(ranks * onehot, axis=1) - 1

    nb = (counts + BLK - 1) // BLK
    nboff = jnp.concatenate(
        [jnp.zeros((1,), jnp.int32), jnp.cumsum(nb)[:2].astype(jnp.int32)])

    j = jnp.arange(NBLK, dtype=jnp.int32)
    bt = ((j >= nboff[1]).astype(jnp.int32)
          + (j >= nboff[2]).astype(jnp.int32))

    inv = nboff[pred] * BLK + rank
    nbtot = nboff[2] + nb[2]
    av = (j < nbtot).astype(jnp.int32)
    return inv, bt, av


def _sc_dispatch(table, inv2d, rowids, rowids2d):
    d = table.shape[1]
    per_w = NSLOT // NW
    nrow_s = (B // 128) // 16
    sl_per_s = NSLOT // 16
    chunks = [80, 64]
    assert sum(chunks) == per_w
    mesh = plsc.VectorSubcoreMesh(core_axis_name="c", subcore_axis_name="s")

    @functools.partial(
        pl.kernel,
        out_type=jax.ShapeDtypeStruct((NSLOT, d), table.dtype),
        mesh=mesh,
        scratch_types=[
            pltpu.VMEM((nrow_s, 128), jnp.int32),
            pltpu.VMEM((nrow_s, 128), jnp.int32),
            pltpu.VMEM((sl_per_s,), jnp.int32),
            pltpu.VMEM((max(chunks),), jnp.int32),
            pltpu.VMEM((max(chunks), d), table.dtype),
            pltpu.VMEM_SHARED((NSLOT,), jnp.int32),
            pltpu.SemaphoreType.DMA,
        ],
    )
    def k(table_hbm, inv_hbm, rowid_hbm, rowid2_hbm, out_hbm,
          inv_v, rid_v, init_v, idx_v, rows_v, slotmap_sh, sem):
        cid = lax.axis_index("c")
        sid = lax.axis_index("s")
        wid = sid * 2 + cid

        sbase = sid * sl_per_s
        pltpu.sync_copy(rowid_hbm.at[pl.ds(sbase, sl_per_s)], init_v)
        pltpu.sync_copy(init_v, slotmap_sh.at[pl.ds(sbase, sl_per_s)])

        rbase = sid * nrow_s
        pltpu.sync_copy(inv_hbm.at[pl.ds(rbase, nrow_s)], inv_v)
        pltpu.sync_copy(rowid2_hbm.at[pl.ds(rbase, nrow_s)], rid_v)
        for h in range(nrow_s):
            pltpu.sync_copy(rid_v.at[h], slotmap_sh.at[inv_v.at[h]])
        plsc.subcore_barrier()

        base = wid * per_w
        coff = 0
        for ch in chunks:
            off = base + coff
            iv = idx_v.at[pl.ds(0, ch)]
            rv = rows_v.at[pl.ds(0, ch)]
            pltpu.sync_copy(slotmap_sh.at[pl.ds(off, ch)], iv)
            pltpu.async_copy(table_hbm.at[iv], rv, sem).wait()
            pltpu.sync_copy(rv, out_hbm.at[pl.ds(off, ch)])
            coff += ch

    return k(table, inv2d, rowids, rowids2d)


def _sc_gather(table, idx, chunks):
    n, d = idx.shape[0], table.shape[1]
    per_w = n // NW
    assert sum(chunks) == per_w
    cmax = max(chunks)
    mesh = plsc.VectorSubcoreMesh(core_axis_name="c", subcore_axis_name="s")

    @functools.partial(
        pl.kernel,
        out_type=jax.ShapeDtypeStruct((n, d), table.dtype),
        mesh=mesh,
        scratch_types=[
            pltpu.VMEM((cmax,), jnp.int32),
            pltpu.VMEM((cmax, d), table.dtype),
            pltpu.SemaphoreType.DMA,
        ],
    )
    def k(table_hbm, idx_hbm, out_hbm, idx_v, rows_v, sem):
        wid = lax.axis_index("s") * 2 + lax.axis_index("c")
        base = wid * per_w
        coff = 0
        for ch in chunks:
            off = base + coff
            iv = idx_v.at[pl.ds(0, ch)]
            rv = rows_v.at[pl.ds(0, ch)]
            pltpu.sync_copy(idx_hbm.at[pl.ds(off, ch)], iv)
            pltpu.async_copy(table_hbm.at[iv], rv, sem).wait()
            pltpu.sync_copy(rv, out_hbm.at[pl.ds(off, ch)])
            coff += ch

    return k(table, idx)


def _mlp_body(bt_ref, av_ref, x_ref,
              w1a, w1b, w1c, b1a, b1b, b1c,
              w2a, w2b, w2c, b2a, b2b, b2c,
              o_ref, h_ref):
    i = pl.program_id(0)
    t = bt_ref[i]
    active = av_ref[i] == 1

    def do_h(w1, b1):
        def _():
            h_ref[...] = jnp.tanh(
                jnp.dot(x_ref[...], w1[...],
                        preferred_element_type=jnp.float32)
                + b1[...])
        return _

    pl.when(active & (t == 0))(do_h(w1a, b1a))
    pl.when(active & (t == 1))(do_h(w1b, b1b))
    pl.when(active & (t == 2))(do_h(w1c, b1c))

    @pl.when(active & (t == 2))
    def _():
        o_ref[:, :D_OUT] = (
            jnp.dot(h_ref[...], w2c[...],
                    preferred_element_type=jnp.float32) + b2c[...])

    @pl.when(active & (t < 2))
    def _():
        w2s = jnp.where(t == 0, w2a[...], w2b[...])
        b2s = jnp.where(t == 0, b2a[...], b2b[...])
        o_ref[:, :NSMALL] = (
            jnp.dot(h_ref[...], w2s, preferred_element_type=jnp.float32)
            + b2s)
        o_ref[:, NSMALL:D_OUT] = jnp.zeros((BLK, D_OUT - NSMALL), jnp.float32)


def _mlp(bt, av, x_sorted, ws):
    sblock = lambda shape: pl.BlockSpec(shape, lambda i, bt, av: (0,) * len(shape))
    grid_spec = pltpu.PrefetchScalarGridSpec(
        num_scalar_prefetch=2,
        grid=(NBLK,),
        in_specs=[
            pl.BlockSpec((BLK, D_IN), lambda i, bt, av: (i, 0)),
            *[sblock(w.shape) for w in ws],
        ],
        out_specs=pl.BlockSpec((BLK, D_PAD), lambda i, bt, av: (i, 0)),
        scratch_shapes=[pltpu.VMEM((BLK, D_HID), jnp.float32)],
    )
    return pl.pallas_call(
        _mlp_body,
        grid_spec=grid_spec,
        out_shape=jax.ShapeDtypeStruct((NSLOT, D_PAD), jnp.float32),
    )(bt, av, x_sorted, *ws)


def kernel(hidden_states, question_type_output,
           W1_yn, b1_yn, W2_yn, b2_yn,
           W1_num, b1_num, W2_num, b2_num,
           W1_oth, b1_oth, W2_oth, b2_oth):
    inv, bt, av = _routing_metadata(question_type_output)

    rowids = jnp.arange(NSLOT, dtype=jnp.int32) % B
    x_sorted = _sc_dispatch(hidden_states, inv.reshape(B // 128, 128),
                            rowids, rowids[:B].reshape(B // 128, 128))

    def pad_small(w2):
        return jnp.pad(w2, ((0, 0), (0, NSMALL - w2.shape[1])))

    def pad_small_b(b2):
        return jnp.pad(b2, ((0, NSMALL - b2.shape[0]),)).reshape(1, NSMALL)

    ws = [
        W1_yn, W1_num, W1_oth,
        b1_yn.reshape(1, D_HID), b1_num.reshape(1, D_HID),
        b1_oth.reshape(1, D_HID),
        pad_small(W2_yn), pad_small(W2_num), W2_oth,
        pad_small_b(b2_yn), pad_small_b(b2_num), b2_oth.reshape(1, D_OUT),
    ]
    y_sorted = _mlp(bt, av, x_sorted, ws)

    return _sc_gather(y_sorted, inv, chunks=[80, 48])[:, :D_OUT]


# speedup vs baseline: 1.0360x; 1.0360x over previous
import functools

import jax
import jax.numpy as jnp
from jax import lax
from jax.experimental import pallas as pl
from jax.experimental.pallas import tpu as pltpu
from jax.experimental.pallas import tpu_sc as plsc

B = 4096
D_IN = 1024
D_HID = 1000
D_OUT = 1000
D_PAD = 1024
BLK = 512
NBLK = B // BLK + 2
NSLOT = NBLK * BLK
NSMALL = 128
NW = 32


def _routing_metadata(question_type_output):
    q = question_type_output
    pred = jnp.argmax(q, axis=1).astype(jnp.int32)
    onehot = (pred[:, None] == jnp.arange(3, dtype=jnp.int32)[None, :])
    ranks = jnp.cumsum(onehot.astype(jnp.int32), axis=0)
    counts = ranks[-1]
    rank = jnp.sum(ranks * onehot, axis=1) - 1

    nb = (counts + BLK - 1) // BLK
    nboff = jnp.concatenate(
        [jnp.zeros((1,), jnp.int32), jnp.cumsum(nb)[:2].astype(jnp.int32)])

    j = jnp.arange(NBLK, dtype=jnp.int32)
    bt = ((j >= nboff[1]).astype(jnp.int32)
          + (j >= nboff[2]).astype(jnp.int32))

    inv = nboff[pred] * BLK + rank
    nbtot = nboff[2] + nb[2]
    av = (j < nbtot).astype(jnp.int32)
    return inv, bt, av


def _sc_dispatch(table, inv2d, rowids, rowids2d):
    d = table.shape[1]
    per_w = NSLOT // NW
    nrow_s = (B // 128) // 16
    sl_per_s = NSLOT // 16
    chunks = [80, 80]
    assert sum(chunks) == per_w
    mesh = plsc.VectorSubcoreMesh(core_axis_name="c", subcore_axis_name="s")

    @functools.partial(
        pl.kernel,
        out_type=jax.ShapeDtypeStruct((NSLOT, d), table.dtype),
        mesh=mesh,
        scratch_types=[
            pltpu.VMEM((nrow_s, 128), jnp.int32),
            pltpu.VMEM((nrow_s, 128), jnp.int32),
            pltpu.VMEM((sl_per_s,), jnp.int32),
            pltpu.VMEM((max(chunks),), jnp.int32),
            pltpu.VMEM((max(chunks), d), table.dtype),
            pltpu.VMEM_SHARED((NSLOT,), jnp.int32),
            pltpu.SemaphoreType.DMA,
        ],
    )
    def k(table_hbm, inv_hbm, rowid_hbm, rowid2_hbm, out_hbm,
          inv_v, rid_v, init_v, idx_v, rows_v, slotmap_sh, sem):
        cid = lax.axis_index("c")
        sid = lax.axis_index("s")
        wid = sid * 2 + cid

        sbase = sid * sl_per_s
        pltpu.sync_copy(rowid_hbm.at[pl.ds(sbase, sl_per_s)], init_v)
        pltpu.sync_copy(init_v, slotmap_sh.at[pl.ds(sbase, sl_per_s)])

        rbase = sid * nrow_s
        pltpu.sync_copy(inv_hbm.at[pl.ds(rbase, nrow_s)], inv_v)
        pltpu.sync_copy(rowid2_hbm.at[pl.ds(rbase, nrow_s)], rid_v)
        for h in range(nrow_s):
            pltpu.sync_copy(rid_v.at[h], slotmap_sh.at[inv_v.at[h]])
        plsc.subcore_barrier()

        base = wid * per_w
        coff = 0
        for ch in chunks:
            off = base + coff
            iv = idx_v.at[pl.ds(0, ch)]
            rv = rows_v.at[pl.ds(0, ch)]
            pltpu.sync_copy(slotmap_sh.at[pl.ds(off, ch)], iv)
            pltpu.async_copy(table_hbm.at[iv], rv, sem).wait()
            pltpu.sync_copy(rv, out_hbm.at[pl.ds(off, ch)])
            coff += ch

    return k(table, inv2d, rowids, rowids2d)


def _sc_gather(table, idx, chunks):
    n, d = idx.shape[0], table.shape[1]
    per_w = n // NW
    assert sum(chunks) == per_w
    cmax = max(chunks)
    mesh = plsc.VectorSubcoreMesh(core_axis_name="c", subcore_axis_name="s")

    @functools.partial(
        pl.kernel,
        out_type=jax.ShapeDtypeStruct((n, d), table.dtype),
        mesh=mesh,
        scratch_types=[
            pltpu.VMEM((cmax,), jnp.int32),
            pltpu.VMEM((cmax, d), table.dtype),
            pltpu.SemaphoreType.DMA,
        ],
    )
    def k(table_hbm, idx_hbm, out_hbm, idx_v, rows_v, sem):
        wid = lax.axis_index("s") * 2 + lax.axis_index("c")
        base = wid * per_w
        coff = 0
        for ch in chunks:
            off = base + coff
            iv = idx_v.at[pl.ds(0, ch)]
            rv = rows_v.at[pl.ds(0, ch)]
            pltpu.sync_copy(idx_hbm.at[pl.ds(off, ch)], iv)
            pltpu.async_copy(table_hbm.at[iv], rv, sem).wait()
            pltpu.sync_copy(rv, out_hbm.at[pl.ds(off, ch)])
            coff += ch

    return k(table, idx)


def _mlp_body(bt_ref, av_ref, x_ref,
              w1a, w1b, w1c, b1a, b1b, b1c,
              w2a, w2b, w2c, b2a, b2b, b2c,
              o_ref, h_ref):
    i = pl.program_id(0)
    t = bt_ref[i]
    active = av_ref[i] == 1

    def do_h(w1, b1):
        def _():
            h_ref[...] = jnp.tanh(
                jnp.dot(x_ref[...], w1[...],
                        preferred_element_type=jnp.float32)
                + b1[...])
        return _

    pl.when(active & (t == 0))(do_h(w1a, b1a))
    pl.when(active & (t == 1))(do_h(w1b, b1b))
    pl.when(active & (t == 2))(do_h(w1c, b1c))

    @pl.when(active & (t == 2))
    def _():
        o_ref[:, :D_OUT] = (
            jnp.dot(h_ref[...], w2c[...],
                    preferred_element_type=jnp.float32) + b2c[...])

    @pl.when(active & (t < 2))
    def _():
        w2s = jnp.where(t == 0, w2a[...], w2b[...])
        b2s = jnp.where(t == 0, b2a[...], b2b[...])
        o_ref[:, :NSMALL] = (
            jnp.dot(h_ref[...], w2s, preferred_element_type=jnp.float32)
            + b2s)
        o_ref[:, NSMALL:D_OUT] = jnp.zeros((BLK, D_OUT - NSMALL), jnp.float32)


def _mlp(bt, av, x_sorted, ws):
    sblock = lambda shape: pl.BlockSpec(shape, lambda i, bt, av: (0,) * len(shape))
    grid_spec = pltpu.PrefetchScalarGridSpec(
        num_scalar_prefetch=2,
        grid=(NBLK,),
        in_specs=[
            pl.BlockSpec((BLK, D_IN), lambda i, bt, av: (i, 0)),
            *[sblock(w.shape) for w in ws],
        ],
        out_specs=pl.BlockSpec((BLK, D_PAD), lambda i, bt, av: (i, 0)),
        scratch_shapes=[pltpu.VMEM((BLK, D_HID), jnp.float32)],
    )
    return pl.pallas_call(
        _mlp_body,
        grid_spec=grid_spec,
        out_shape=jax.ShapeDtypeStruct((NSLOT, D_PAD), jnp.float32),
    )(bt, av, x_sorted, *ws)


def kernel(hidden_states, question_type_output,
           W1_yn, b1_yn, W2_yn, b2_yn,
           W1_num, b1_num, W2_num, b2_num,
           W1_oth, b1_oth, W2_oth, b2_oth):
    inv, bt, av = _routing_metadata(question_type_output)

    rowids = jnp.arange(NSLOT, dtype=jnp.int32) % B
    x_sorted = _sc_dispatch(hidden_states, inv.reshape(B // 128, 128),
                            rowids, rowids[:B].reshape(B // 128, 128))

    def pad_small(w2):
        return jnp.pad(w2, ((0, 0), (0, NSMALL - w2.shape[1])))

    def pad_small_b(b2):
        return jnp.pad(b2, ((0, NSMALL - b2.shape[0]),)).reshape(1, NSMALL)

    ws = [
        W1_yn, W1_num, W1_oth,
        b1_yn.reshape(1, D_HID), b1_num.reshape(1, D_HID),
        b1_oth.reshape(1, D_HID),
        pad_small(W2_yn), pad_small(W2_num), W2_oth,
        pad_small_b(b2_yn), pad_small_b(b2_num), b2_oth.reshape(1, D_OUT),
    ]
    y_sorted = _mlp(bt, av, x_sorted, ws)

    return _sc_gather(y_sorted, inv, chunks=[80, 48])[:, :D_OUT]
